# baseline (speedup 1.0000x reference)
import functools

import jax
import jax.numpy as jnp
from jax import lax
from jax.experimental import pallas as pl
from jax.experimental.pallas import tpu as pltpu
from jax.experimental.pallas import tpu_sc as plsc

K = 8192
D = 64
N = 9216
ROWS = 256
NBLK = N // ROWS

NW = 32
B_PER_W = N // NW
CHUNK = 96
NCHUNK = B_PER_W // CHUNK


NWIN = 4
WCOLS = K // NWIN


def _distance_argmin_body(x_ref, w_ref, idx_ref, loss_ref):
    x = x_ref[...]
    x2 = jnp.sum(x * x, axis=1, keepdims=True)
    xb = x.astype(jnp.bfloat16)
    acc_i = acc_b = m_chosen = None
    for c in range(NWIN):
        wc = w_ref[c * WCOLS:(c + 1) * WCOLS, :]
        w2c = jnp.sum(wc * wc, axis=1)
        mm2c = lax.dot_general(xb, (wc * 2.0).astype(jnp.bfloat16),
                               (((1,), (1,)), ((), ())),
                               preferred_element_type=jnp.float32)
        dc = (x2 + w2c[None, :]) - mm2c
        mc = jnp.min(dc, axis=1, keepdims=True)
        ic = jnp.argmin(dc, axis=1).astype(jnp.int32) + jnp.int32(c * WCOLS)
        mcb = mc.astype(jnp.bfloat16).astype(jnp.float32)
        if c == 0:
            acc_i, acc_b, m_chosen = ic, mcb, mc
        else:
            sel = mc < acc_b
            acc_i = jnp.where(sel[:, 0], ic, acc_i)
            acc_b = jnp.where(sel, mcb, acc_b)
            m_chosen = jnp.where(sel, mc, m_chosen)
    idx_ref[...] = acc_i.reshape(1, 1, ROWS)
    m = m_chosen

    @pl.when(pl.program_id(0) == 0)
    def _():
        loss_ref[0, 0] = 0.0

    loss_ref[0, 0] += jnp.sum(m)

    @pl.when(pl.program_id(0) == NBLK - 1)
    def _():
        loss_ref[0, 0] *= 1.25 / (N * D)


_distance_argmin = pl.pallas_call(
    _distance_argmin_body,
    grid=(NBLK,),
    in_specs=[
        pl.BlockSpec((ROWS, D), lambda i: (i, 0)),
        pl.BlockSpec((K, D), lambda i: (0, 0)),
    ],
    out_specs=[
        pl.BlockSpec((1, 1, ROWS), lambda i: (i, 0, 0)),
        pl.BlockSpec(memory_space=pltpu.SMEM),
    ],
    out_shape=[
        jax.ShapeDtypeStruct((NBLK, 1, ROWS), jnp.int32),
        jax.ShapeDtypeStruct((1, 1), jnp.float32),
    ],
)


@functools.partial(
    pl.kernel,
    out_type=jax.ShapeDtypeStruct((N, D), jnp.float32),
    mesh=plsc.VectorSubcoreMesh(core_axis_name="c", subcore_axis_name="s"),
    compiler_params=pltpu.CompilerParams(use_tc_tiling_on_sc=False),
    scratch_types=(
        [pltpu.VMEM((CHUNK,), jnp.int32) for _ in range(NCHUNK)]
        + [pltpu.VMEM((CHUNK, D), jnp.float32) for _ in range(NCHUNK)]
        + [pltpu.SemaphoreType.DMA]
    ),
)
def _gather_rows(table_hbm, idx_hbm, out_hbm, i0, i1, i2, r0, r1, r2, sem):
    wid = lax.axis_index("s") * 2 + lax.axis_index("c")
    base = wid * B_PER_W
    idx_bufs = (i0, i1, i2)
    row_bufs = (r0, r1, r2)
    for c in range(NCHUNK):
        pltpu.sync_copy(idx_hbm.at[pl.ds(base + c * CHUNK, CHUNK)], idx_bufs[c])
    copies = [
        pltpu.async_copy(table_hbm.at[idx_bufs[c]], row_bufs[c], sem)
        for c in range(NCHUNK)
    ]
    for cp in copies:
        cp.wait()
    for c in range(NCHUNK):
        pltpu.sync_copy(row_bufs[c], out_hbm.at[pl.ds(base + c * CHUNK, CHUNK)])


def kernel(inputs, embed_w):
    flat = inputs.reshape(-1, D)
    idx3, loss = _distance_argmin(flat, embed_w)
    idx = idx3.reshape(N)
    quantized = _gather_rows(embed_w, idx).reshape(inputs.shape)
    return quantized, loss[0, 0], idx


# speedup vs baseline: 1.4609x; 1.4609x over previous
import functools

import jax
import jax.numpy as jnp
from jax import lax
from jax.experimental import pallas as pl
from jax.experimental.pallas import tpu as pltpu
from jax.experimental.pallas import tpu_sc as plsc

K = 8192
D = 64
N = 9216
ROWS = 256
NBLK = N // ROWS

NW = 32
B_PER_W = N // NW
CHUNK = 96
NCHUNK = B_PER_W // CHUNK


NWIN = 4
WCOLS = K // NWIN


def _distance_argmin_body(x_ref, w_ref, idx_ref, loss_ref):
    x = x_ref[...]
    x2 = jnp.sum(x * x, axis=1, keepdims=True)
    xb = x.astype(jnp.bfloat16)
    ids = lax.broadcasted_iota(jnp.int32, (ROWS, WCOLS), 1)
    acc_i = acc_b = m_chosen = None
    for c in range(NWIN):
        wc = w_ref[c * WCOLS:(c + 1) * WCOLS, :]
        w2c = jnp.sum(wc * wc, axis=1)
        mm2c = lax.dot_general(xb, (wc * 2.0).astype(jnp.bfloat16),
                               (((1,), (1,)), ((), ())),
                               preferred_element_type=jnp.float32)
        dc = (x2 + w2c[None, :]) - mm2c
        mc = jnp.min(dc, axis=1, keepdims=True)
        ic = jnp.min(jnp.where(dc == mc, ids + c * WCOLS, jnp.int32(K)), axis=1)
        mcb = mc.astype(jnp.bfloat16).astype(jnp.float32)
        if c == 0:
            acc_i, acc_b, m_chosen = ic, mcb, mc
        else:
            sel = mc < acc_b
            acc_i = jnp.where(sel[:, 0], ic, acc_i)
            acc_b = jnp.where(sel, mcb, acc_b)
            m_chosen = jnp.where(sel, mc, m_chosen)
    idx_ref[...] = acc_i.reshape(1, 1, ROWS)
    m = m_chosen

    @pl.when(pl.program_id(0) == 0)
    def _():
        loss_ref[0, 0] = 0.0

    loss_ref[0, 0] += jnp.sum(m)

    @pl.when(pl.program_id(0) == NBLK - 1)
    def _():
        loss_ref[0, 0] *= 1.25 / (N * D)


_distance_argmin = pl.pallas_call(
    _distance_argmin_body,
    grid=(NBLK,),
    in_specs=[
        pl.BlockSpec((ROWS, D), lambda i: (i, 0)),
        pl.BlockSpec((K, D), lambda i: (0, 0)),
    ],
    out_specs=[
        pl.BlockSpec((1, 1, ROWS), lambda i: (i, 0, 0)),
        pl.BlockSpec(memory_space=pltpu.SMEM),
    ],
    out_shape=[
        jax.ShapeDtypeStruct((NBLK, 1, ROWS), jnp.int32),
        jax.ShapeDtypeStruct((1, 1), jnp.float32),
    ],
)


@functools.partial(
    pl.kernel,
    out_type=jax.ShapeDtypeStruct((N, D), jnp.float32),
    mesh=plsc.VectorSubcoreMesh(core_axis_name="c", subcore_axis_name="s"),
    compiler_params=pltpu.CompilerParams(use_tc_tiling_on_sc=False),
    scratch_types=(
        [pltpu.VMEM((CHUNK,), jnp.int32) for _ in range(NCHUNK)]
        + [pltpu.VMEM((CHUNK, D), jnp.float32) for _ in range(NCHUNK)]
        + [pltpu.SemaphoreType.DMA]
    ),
)
def _gather_rows(table_hbm, idx_hbm, out_hbm, i0, i1, i2, r0, r1, r2, sem):
    wid = lax.axis_index("s") * 2 + lax.axis_index("c")
    base = wid * B_PER_W
    idx_bufs = (i0, i1, i2)
    row_bufs = (r0, r1, r2)
    for c in range(NCHUNK):
        pltpu.sync_copy(idx_hbm.at[pl.ds(base + c * CHUNK, CHUNK)], idx_bufs[c])
    copies = [
        pltpu.async_copy(table_hbm.at[idx_bufs[c]], row_bufs[c], sem)
        for c in range(NCHUNK)
    ]
    for cp in copies:
        cp.wait()
    for c in range(NCHUNK):
        pltpu.sync_copy(row_bufs[c], out_hbm.at[pl.ds(base + c * CHUNK, CHUNK)])


def kernel(inputs, embed_w):
    flat = inputs.reshape(-1, D)
    idx3, loss = _distance_argmin(flat, embed_w)
    idx = idx3.reshape(N)
    quantized = _gather_rows(embed_w, idx).reshape(inputs.shape)
    return quantized, loss[0, 0], idx


# speedup vs baseline: 1.8533x; 1.2686x over previous
import functools

import jax
import jax.numpy as jnp
from jax import lax
from jax.experimental import pallas as pl
from jax.experimental.pallas import tpu as pltpu
from jax.experimental.pallas import tpu_sc as plsc

K = 8192
D = 64
N = 9216
ROWS = 1152
NBLK = N // ROWS

NW = 32
B_PER_W = N // NW
CHUNK = 96
NCHUNK = B_PER_W // CHUNK


NWIN = 4
WCOLS = K // NWIN


def _distance_argmin_body(x_ref, w_ref, idx_ref, loss_ref):
    x = x_ref[...]
    x2 = jnp.sum(x * x, axis=1, keepdims=True)
    xb = x.astype(jnp.bfloat16)
    ids = lax.broadcasted_iota(jnp.int32, (ROWS, WCOLS), 1)
    acc_i = acc_b = m_chosen = None
    for c in range(NWIN):
        wc = w_ref[c * WCOLS:(c + 1) * WCOLS, :]
        w2c = jnp.sum(wc * wc, axis=1)
        mm2c = lax.dot_general(xb, (wc * 2.0).astype(jnp.bfloat16),
                               (((1,), (1,)), ((), ())),
                               preferred_element_type=jnp.float32)
        dc = (x2 + w2c[None, :]) - mm2c
        mc = jnp.min(dc, axis=1, keepdims=True)
        ic = jnp.min(jnp.where(dc == mc, ids + c * WCOLS, jnp.int32(K)), axis=1)
        mcb = mc.astype(jnp.bfloat16).astype(jnp.float32)
        if c == 0:
            acc_i, acc_b, m_chosen = ic, mcb, mc
        else:
            sel = mc < acc_b
            acc_i = jnp.where(sel[:, 0], ic, acc_i)
            acc_b = jnp.where(sel, mcb, acc_b)
            m_chosen = jnp.where(sel, mc, m_chosen)
    idx_ref[...] = acc_i.reshape(1, 1, ROWS)
    m = m_chosen

    @pl.when(pl.program_id(0) == 0)
    def _():
        loss_ref[0, 0] = 0.0

    loss_ref[0, 0] += jnp.sum(m)

    @pl.when(pl.program_id(0) == NBLK - 1)
    def _():
        loss_ref[0, 0] *= 1.25 / (N * D)


_distance_argmin = pl.pallas_call(
    _distance_argmin_body,
    grid=(NBLK,),
    in_specs=[
        pl.BlockSpec((ROWS, D), lambda i: (i, 0)),
        pl.BlockSpec((K, D), lambda i: (0, 0)),
    ],
    out_specs=[
        pl.BlockSpec((1, 1, ROWS), lambda i: (i, 0, 0)),
        pl.BlockSpec(memory_space=pltpu.SMEM),
    ],
    out_shape=[
        jax.ShapeDtypeStruct((NBLK, 1, ROWS), jnp.int32),
        jax.ShapeDtypeStruct((1, 1), jnp.float32),
    ],
)


@functools.partial(
    pl.kernel,
    out_type=jax.ShapeDtypeStruct((N, D), jnp.float32),
    mesh=plsc.VectorSubcoreMesh(core_axis_name="c", subcore_axis_name="s"),
    compiler_params=pltpu.CompilerParams(use_tc_tiling_on_sc=False),
    scratch_types=(
        [pltpu.VMEM((CHUNK,), jnp.int32) for _ in range(NCHUNK)]
        + [pltpu.VMEM((CHUNK, D), jnp.float32) for _ in range(NCHUNK)]
        + [pltpu.SemaphoreType.DMA]
    ),
)
def _gather_rows(table_hbm, idx_hbm, out_hbm, i0, i1, i2, r0, r1, r2, sem):
    wid = lax.axis_index("s") * 2 + lax.axis_index("c")
    base = wid * B_PER_W
    idx_bufs = (i0, i1, i2)
    row_bufs = (r0, r1, r2)
    for c in range(NCHUNK):
        pltpu.sync_copy(idx_hbm.at[pl.ds(base + c * CHUNK, CHUNK)], idx_bufs[c])
    copies = [
        pltpu.async_copy(table_hbm.at[idx_bufs[c]], row_bufs[c], sem)
        for c in range(NCHUNK)
    ]
    for cp in copies:
        cp.wait()
    for c in range(NCHUNK):
        pltpu.sync_copy(row_bufs[c], out_hbm.at[pl.ds(base + c * CHUNK, CHUNK)])


def kernel(inputs, embed_w):
    flat = inputs.reshape(-1, D)
    idx3, loss = _distance_argmin(flat, embed_w)
    idx = idx3.reshape(N)
    quantized = _gather_rows(embed_w, idx).reshape(inputs.shape)
    return quantized, loss[0, 0], idx
